# Initial kernel scaffold; baseline (speedup 1.0000x reference)
import functools

import jax
import jax.numpy as jnp
from jax import lax
from jax.experimental import pallas as pl
from jax.experimental.pallas import tpu as pltpu
from jax.experimental.pallas import tpu_sc as plsc

N_NODES = 10000
N_EDGES = 320000
D = 128

NC = 2
NS = 16
NW = NC * NS

CHUNK = 128
CHUNKS_PER_TILE = 80
EPT = CHUNK * CHUNKS_PER_TILE
E_PAD = EPT * NW
N_PAD = 10016
RPT = N_PAD // NS

_mesh = plsc.VectorSubcoreMesh(core_axis_name="c", subcore_axis_name="s")


def _make_agg_kernel(with_deg: bool):
    out_type = [jax.ShapeDtypeStruct((NC, N_PAD, D), jnp.float32)]
    scratch = [
        pltpu.VMEM_SHARED((N_PAD, D), jnp.float32),
        pltpu.VMEM((CHUNKS_PER_TILE, CHUNK), jnp.int32),
        pltpu.VMEM((CHUNKS_PER_TILE, CHUNK), jnp.int32),
        pltpu.VMEM((CHUNK, D), jnp.float32),
        pltpu.SemaphoreType.DMA,
    ]
    if with_deg:
        out_type.append(jax.ShapeDtypeStruct((NC, N_PAD, 16), jnp.float32))
        scratch += [
            pltpu.VMEM_SHARED((N_PAD, 16), jnp.float32),
            pltpu.VMEM((CHUNK, 16), jnp.float32),
        ]

    def body(*refs):
        if with_deg:
            (h_hbm, src_hbm, dst_hbm, zagg_hbm, zdeg_hbm, ones_hbm,
             agg_out, deg_out,
             agg_sh, src_v, dst_v, rows_v, sem, deg_sh, ones_v) = refs
        else:
            (h_hbm, src_hbm, dst_hbm, zagg_hbm,
             agg_out,
             agg_sh, src_v, dst_v, rows_v, sem) = refs
        c = lax.axis_index("c")
        s = lax.axis_index("s")
        wid = c * NS + s

        pltpu.sync_copy(zagg_hbm.at[pl.ds(s * RPT, RPT)],
                        agg_sh.at[pl.ds(s * RPT, RPT)])
        if with_deg:
            pltpu.sync_copy(zdeg_hbm.at[pl.ds(s * RPT, RPT)],
                            deg_sh.at[pl.ds(s * RPT, RPT)])
            pltpu.sync_copy(ones_hbm, ones_v)

        pltpu.sync_copy(src_hbm.at[pl.ds(wid * CHUNKS_PER_TILE, CHUNKS_PER_TILE)],
                        src_v)
        pltpu.sync_copy(dst_hbm.at[pl.ds(wid * CHUNKS_PER_TILE, CHUNKS_PER_TILE)],
                        dst_v)
        plsc.subcore_barrier()

        def chunk_body(j, carry):
            pltpu.async_copy(h_hbm.at[src_v.at[j]], rows_v, sem).wait()
            pltpu.sync_copy(rows_v, agg_sh.at[dst_v.at[j]], add=True)
            if with_deg:
                pltpu.sync_copy(ones_v, deg_sh.at[dst_v.at[j]], add=True)
            return carry

        lax.fori_loop(0, CHUNKS_PER_TILE, chunk_body, 0)
        plsc.subcore_barrier()

        pltpu.sync_copy(agg_sh.at[pl.ds(s * RPT, RPT)],
                        agg_out.at[c, pl.ds(s * RPT, RPT)])
        if with_deg:
            pltpu.sync_copy(deg_sh.at[pl.ds(s * RPT, RPT)],
                            deg_out.at[c, pl.ds(s * RPT, RPT)])

    return functools.partial(pl.kernel, mesh=_mesh, out_type=out_type,
                             scratch_types=scratch)(body)


_agg_deg_kernel = _make_agg_kernel(with_deg=True)
_agg_kernel = _make_agg_kernel(with_deg=False)


BR = 400
GRID = N_NODES // BR


def _dense_body(aggp, degp, h, wm, ws, b, out):
    p = aggp[0] + aggp[1]
    deg = jnp.maximum(degp[0, :, 0:1] + degp[1, :, 0:1], 1.0)
    agg = p / deg
    out[...] = jnp.maximum(
        jnp.dot(agg, wm[...], preferred_element_type=jnp.float32)
        + jnp.dot(h[...], ws[...], preferred_element_type=jnp.float32)
        + b[...], 0.0)


def _dense_pool_body(aggp, degp, h, wm, ws, b, out):
    i = pl.program_id(0)
    p = aggp[0] + aggp[1]
    deg = jnp.maximum(degp[0, :, 0:1] + degp[1, :, 0:1], 1.0)
    agg = p / deg
    hn = jnp.maximum(
        jnp.dot(agg, wm[...], preferred_element_type=jnp.float32)
        + jnp.dot(h[...], ws[...], preferred_element_type=jnp.float32)
        + b[...], 0.0)
    part = jnp.sum(hn, axis=0, keepdims=True) * (1.0 / N_NODES)

    @pl.when(i == 0)
    def _init():
        out[...] = part

    @pl.when(i != 0)
    def _acc():
        out[...] = out[...] + part


_dense_specs = dict(
    grid=(GRID,),
    in_specs=[
        pl.BlockSpec((NC, BR, D), lambda i: (0, i, 0)),
        pl.BlockSpec((NC, BR, 16), lambda i: (0, i, 0)),
        pl.BlockSpec((BR, D), lambda i: (i, 0)),
        pl.BlockSpec((D, D), lambda i: (0, 0)),
        pl.BlockSpec((D, D), lambda i: (0, 0)),
        pl.BlockSpec((1, D), lambda i: (0, 0)),
    ],
    compiler_params=pltpu.CompilerParams(
        dimension_semantics=("arbitrary",)),
)

_dense_layer = pl.pallas_call(
    _dense_body,
    out_shape=jax.ShapeDtypeStruct((N_NODES, D), jnp.float32),
    out_specs=pl.BlockSpec((BR, D), lambda i: (i, 0)),
    **_dense_specs,
)

_dense_pool_layer = pl.pallas_call(
    _dense_pool_body,
    out_shape=jax.ShapeDtypeStruct((1, D), jnp.float32),
    out_specs=pl.BlockSpec((1, D), lambda i: (0, 0)),
    **_dense_specs,
)


def kernel(x, edge_index, W_msg1, W_self1, b1, W_msg2, W_self2, b2):
    src = edge_index[0].astype(jnp.int32)
    dst = edge_index[1].astype(jnp.int32)
    pad = E_PAD - N_EDGES
    srcp = jnp.concatenate([src, jnp.zeros((pad,), jnp.int32)])
    dstp = jnp.concatenate([dst, jnp.full((pad,), N_NODES, jnp.int32)])
    srcp = srcp.reshape(E_PAD // CHUNK, CHUNK)
    dstp = dstp.reshape(E_PAD // CHUNK, CHUNK)
    zagg = jnp.zeros((N_PAD, D), jnp.float32)
    zdeg = jnp.zeros((N_PAD, 16), jnp.float32)
    ones = jnp.ones((CHUNK, 16), jnp.float32)
    b1r = b1.reshape(1, D)
    b2r = b2.reshape(1, D)

    aggp1, degp = _agg_deg_kernel(x, srcp, dstp, zagg, zdeg, ones)
    h1 = _dense_layer(aggp1, degp, x, W_msg1, W_self1, b1r)
    aggp2 = _agg_kernel(h1, srcp, dstp, zagg)
    out = _dense_pool_layer(aggp2, degp, h1, W_msg2, W_self2, b2r)
    return out


# speedup vs baseline: 2.4552x; 2.4552x over previous
import functools

import jax
import jax.numpy as jnp
from jax import lax
from jax.experimental import pallas as pl
from jax.experimental.pallas import tpu as pltpu
from jax.experimental.pallas import tpu_sc as plsc

N_NODES = 10000
N_EDGES = 320000
D = 128

NC = 2
NS = 16
NW = NC * NS

CHUNK = 128
CHUNKS_PER_TILE = 80
GRP = 8
EPT = CHUNK * CHUNKS_PER_TILE
E_PAD = EPT * NW
N_PAD = 10112
RPT = N_PAD // NS

_mesh = plsc.VectorSubcoreMesh(core_axis_name="c", subcore_axis_name="s")


def _make_agg_kernel():
    out_type = [jax.ShapeDtypeStruct((NC, N_PAD, D), jnp.float32)]
    scratch = [
        pltpu.VMEM_SHARED((N_PAD, D), jnp.float32),
        pltpu.VMEM((GRP, CHUNK), jnp.int32),
        pltpu.VMEM((GRP, CHUNK), jnp.int32),
        pltpu.VMEM((CHUNK, D), jnp.float32),
        pltpu.SemaphoreType.DMA,
    ]

    def body(h_hbm, src_hbm, dst_hbm, zagg_hbm, agg_out,
             agg_sh, src_v, dst_v, rows_v, sem):
        c = lax.axis_index("c")
        s = lax.axis_index("s")
        wid = c * NS + s

        pltpu.sync_copy(zagg_hbm.at[pl.ds(s * RPT, RPT)],
                        agg_sh.at[pl.ds(s * RPT, RPT)])
        plsc.subcore_barrier()

        def grp_body(g, carry):
            base = wid * CHUNKS_PER_TILE + g * GRP
            pltpu.sync_copy(src_hbm.at[pl.ds(base, GRP)], src_v)
            pltpu.sync_copy(dst_hbm.at[pl.ds(base, GRP)], dst_v)

            def chunk_body(j, c2):
                pltpu.async_copy(h_hbm.at[src_v.at[j]], rows_v, sem).wait()
                pltpu.sync_copy(rows_v, agg_sh.at[dst_v.at[j]], add=True)
                return c2

            lax.fori_loop(0, GRP, chunk_body, 0)
            return carry

        lax.fori_loop(0, CHUNKS_PER_TILE // GRP, grp_body, 0)
        plsc.subcore_barrier()

        pltpu.sync_copy(agg_sh.at[pl.ds(s * RPT, RPT)],
                        agg_out.at[c, pl.ds(s * RPT, RPT)])

    return functools.partial(pl.kernel, mesh=_mesh, out_type=out_type,
                             scratch_types=scratch)(body)


_agg_kernel = _make_agg_kernel()


BR = 400
GRID = N_NODES // BR


def _dense_body(aggp, degp, h, wm, ws, b, out):
    p = aggp[0] + aggp[1]
    deg = jnp.maximum(degp[0] + degp[1], 1.0)
    agg = p / deg
    out[...] = jnp.maximum(
        jnp.dot(agg, wm[...], preferred_element_type=jnp.float32)
        + jnp.dot(h[...], ws[...], preferred_element_type=jnp.float32)
        + b[...], 0.0)


def _dense_pool_body(aggp, degp, h, wm, ws, b, out):
    i = pl.program_id(0)
    p = aggp[0] + aggp[1]
    deg = jnp.maximum(degp[0] + degp[1], 1.0)
    agg = p / deg
    hn = jnp.maximum(
        jnp.dot(agg, wm[...], preferred_element_type=jnp.float32)
        + jnp.dot(h[...], ws[...], preferred_element_type=jnp.float32)
        + b[...], 0.0)
    part = jnp.sum(hn, axis=0, keepdims=True) * (1.0 / N_NODES)

    @pl.when(i == 0)
    def _init():
        out[...] = part

    @pl.when(i != 0)
    def _acc():
        out[...] = out[...] + part


_dense_specs = dict(
    grid=(GRID,),
    in_specs=[
        pl.BlockSpec((NC, BR, D), lambda i: (0, i, 0)),
        pl.BlockSpec((NC, BR, D), lambda i: (0, i, 0)),
        pl.BlockSpec((BR, D), lambda i: (i, 0)),
        pl.BlockSpec((D, D), lambda i: (0, 0)),
        pl.BlockSpec((D, D), lambda i: (0, 0)),
        pl.BlockSpec((1, D), lambda i: (0, 0)),
    ],
    compiler_params=pltpu.CompilerParams(
        dimension_semantics=("arbitrary",)),
)

_dense_layer = pl.pallas_call(
    _dense_body,
    out_shape=jax.ShapeDtypeStruct((N_NODES, D), jnp.float32),
    out_specs=pl.BlockSpec((BR, D), lambda i: (i, 0)),
    **_dense_specs,
)

_dense_pool_layer = pl.pallas_call(
    _dense_pool_body,
    out_shape=jax.ShapeDtypeStruct((1, D), jnp.float32),
    out_specs=pl.BlockSpec((1, D), lambda i: (0, 0)),
    **_dense_specs,
)


def kernel(x, edge_index, W_msg1, W_self1, b1, W_msg2, W_self2, b2):
    src = edge_index[0].astype(jnp.int32)
    dst = edge_index[1].astype(jnp.int32)
    pad = E_PAD - N_EDGES
    srcp = jnp.concatenate([src, jnp.zeros((pad,), jnp.int32)])
    dstp = jnp.concatenate([dst, jnp.full((pad,), N_NODES, jnp.int32)])
    srcp = srcp.reshape(E_PAD // CHUNK, CHUNK)
    dstp = dstp.reshape(E_PAD // CHUNK, CHUNK)
    zagg = jnp.zeros((N_PAD, D), jnp.float32)
    b1r = b1.reshape(1, D)
    b2r = b2.reshape(1, D)

    (aggp1,) = _agg_kernel(x, srcp, dstp, zagg)
    (degp,) = _agg_kernel(jnp.ones((N_NODES, D), jnp.float32), srcp, dstp, zagg)
    h1 = _dense_layer(aggp1, degp, x, W_msg1, W_self1, b1r)
    (aggp2,) = _agg_kernel(h1, srcp, dstp, zagg)
    out = _dense_pool_layer(aggp2, degp, h1, W_msg2, W_self2, b2r)
    return out
